# baseline (speedup 1.0000x reference)
import functools

import jax
import jax.numpy as jnp
from jax import lax
from jax.experimental import pallas as pl
from jax.experimental.pallas import tpu as pltpu
from jax.experimental.pallas import tpu_sc as plsc

B, S, D = 4, 8192, 768
DINUC_DIM = D // 4
EPS = 1e-12
TOK = B * S
NC, NS = 2, 16
NW = NC * NS
BPW = TOK // NW
CH = 64
NCH = BPW // CH


def _prep_body(ids_ref, tt_ref, dt_ref, w_ref, b_ref, g_ref, be_ref,
               lut_ref, c_ref):
    w_top = w_ref[:D, :]
    w_bot = w_ref[D:, :]
    tf = jnp.dot(tt_ref[...], w_top, preferred_element_type=jnp.float32)
    df = jnp.dot(dt_ref[...], w_bot, preferred_element_type=jnp.float32)
    df32 = jnp.concatenate([df, jnp.zeros((16, D), jnp.float32)], axis=0)
    x = tf[:, None, :] + df32[None, :, :] + b_ref[...][None, :, :]
    mean = jnp.mean(x, axis=-1, keepdims=True)
    var = jnp.mean((x - mean) ** 2, axis=-1, keepdims=True)
    lut_ref[...] = ((x - mean) * lax.rsqrt(var + EPS)
                    * g_ref[...][None, :, :] + be_ref[...][None, :, :])

    first = ids_ref[...]
    second = jnp.concatenate(
        [first[:, 1:], jnp.zeros((B, 1), jnp.int32)], axis=1)
    valid = ((first >= 4) & (first <= 7) & (second >= 4) & (second <= 7))
    did = jnp.where(valid, (first - 4) * 4 + (second - 4), 0)
    col = lax.broadcasted_iota(jnp.int32, (B, S), 1)
    did = jnp.where(col == S - 1, 16, did)
    c_ref[...] = first * 32 + did


def _prep(input_ids, token_table, dinuc_table, proj_w, proj_b, ln_gamma,
          ln_beta):
    return pl.pallas_call(
        _prep_body,
        out_shape=(
            jax.ShapeDtypeStruct((8, 32, D), jnp.float32),
            jax.ShapeDtypeStruct((B, S), jnp.int32),
        ),
    )(input_ids, token_table, dinuc_table, proj_w,
      proj_b.reshape(1, D), ln_gamma.reshape(1, D), ln_beta.reshape(1, D))


PW = D // 2
NROW = 8 * 32
TT = TOK // NW
GT = 16
NG = TT // GT
OBW = GT * D
NCHAIN = 8


def _fill_block(idx_v, lut_v, ob, g, phase, iota16):
    c_vec = idx_v[pl.ds(g * GT, GT)]
    ob_base = iota16 * D + phase * OBW
    la = [c_vec * PW + b for b in range(NCHAIN)]
    se = [ob_base + 2 * b for b in range(NCHAIN)]
    so = [ob_base + (2 * b + 1) for b in range(NCHAIN)]
    dla = jnp.full((GT,), NCHAIN, jnp.int32)
    dse = jnp.full((GT,), 2 * NCHAIN, jnp.int32)
    for _ in range(PW // NCHAIN):
        for b in range(NCHAIN):
            w = plsc.load_gather(lut_v, [la[b]])
            p0, p1 = plsc.unpack(plsc.bitcast(w, jnp.bfloat16),
                                 format=plsc.PackFormat.INTERLEAVED)
            plsc.store_scatter(ob, [se[b]], p0)
            plsc.store_scatter(ob, [so[b]], p1)
        for b in range(NCHAIN):
            la[b] = la[b] + dla
            se[b] = se[b] + dse
            so[b] = so[b] + dse


def _sc_gather_body(lut_hbm, idx_hbm, out_hbm, idx_v, lut_v, ob, sem0, sem1):
    wid = lax.axis_index("s") * NC + lax.axis_index("c")
    tok0 = wid * TT
    pltpu.sync_copy(lut_hbm, lut_v)
    pltpu.sync_copy(idx_hbm.at[pl.ds(tok0, TT)], idx_v)
    iota16 = lax.iota(jnp.int32, GT)

    def wcopy(i, phase, sem):
        return pltpu.make_async_copy(
            ob.at[pl.ds(phase * OBW, OBW)],
            out_hbm.at[pl.ds((tok0 + i * GT) * D, OBW)], sem)

    def body(i, carry):
        phase = lax.rem(i, 2)
        @pl.when(jnp.logical_and(i >= 2, phase == 0))
        def _():
            wcopy(i, 0, sem0).wait()
        @pl.when(jnp.logical_and(i >= 2, phase == 1))
        def _():
            wcopy(i, 1, sem1).wait()
        _fill_block(idx_v, lut_v, ob, i, phase, iota16)
        @pl.when(phase == 0)
        def _():
            wcopy(i, 0, sem0).start()
        @pl.when(phase == 1)
        def _():
            wcopy(i, 1, sem1).start()
        return carry

    lax.fori_loop(0, NG, body, 0)
    wcopy(0, 0, sem0).wait()
    wcopy(0, 1, sem1).wait()


@functools.cache
def _sc_gather():
    return pl.kernel(
        _sc_gather_body,
        out_type=jax.ShapeDtypeStruct((TOK * D,), jnp.float32),
        mesh=plsc.VectorSubcoreMesh(core_axis_name="c", subcore_axis_name="s",
                                    num_cores=NC, num_subcores=NS),
        scratch_types=[
            pltpu.VMEM((TT,), jnp.int32),
            pltpu.VMEM((NROW * PW,), jnp.int32),
            pltpu.VMEM((2 * OBW,), jnp.float32),
            pltpu.SemaphoreType.DMA,
            pltpu.SemaphoreType.DMA,
        ],
        compiler_params=pltpu.CompilerParams(use_tc_tiling_on_sc=False,
                                             needs_layout_passes=False),
    )


@jax.jit
def kernel(input_ids, token_table, dinuc_table, proj_w, proj_b, ln_gamma,
           ln_beta):
    lut, c = _prep(input_ids, token_table, dinuc_table, proj_w, proj_b,
                   ln_gamma, ln_beta)
    lut_pk = lax.bitcast_convert_type(
        lut.astype(jnp.bfloat16).reshape(NROW, PW, 2), jnp.int32)
    out = _sc_gather()(lut_pk.reshape(NROW * PW), c.reshape(TOK))
    return out.reshape(B, S, D)


# speedup vs baseline: 1.4157x; 1.4157x over previous
import functools

import jax
import jax.numpy as jnp
from jax import lax
from jax.experimental import pallas as pl
from jax.experimental.pallas import tpu as pltpu
from jax.experimental.pallas import tpu_sc as plsc

B, S, D = 4, 8192, 768
DINUC_DIM = D // 4
EPS = 1e-12
TOK = B * S
NC, NS = 2, 16
NW = NC * NS
BPW = TOK // NW
CH = 64
NCH = BPW // CH


def _prep_body(ids_ref, tt_ref, dt_ref, w_ref, b_ref, g_ref, be_ref,
               lut_ref, c_ref):
    w_top = w_ref[:D, :]
    w_bot = w_ref[D:, :]
    tf = jnp.dot(tt_ref[...], w_top, preferred_element_type=jnp.float32)
    df = jnp.dot(dt_ref[...], w_bot, preferred_element_type=jnp.float32)
    df32 = jnp.concatenate([df, jnp.zeros((16, D), jnp.float32)], axis=0)
    x = tf[:, None, :] + df32[None, :, :] + b_ref[...][None, :, :]
    mean = jnp.mean(x, axis=-1, keepdims=True)
    var = jnp.mean((x - mean) ** 2, axis=-1, keepdims=True)
    lut_ref[...] = ((x - mean) * lax.rsqrt(var + EPS)
                    * g_ref[...][None, :, :] + be_ref[...][None, :, :])

    first = ids_ref[...]
    second = jnp.concatenate(
        [first[:, 1:], jnp.zeros((B, 1), jnp.int32)], axis=1)
    valid = ((first >= 4) & (first <= 7) & (second >= 4) & (second <= 7))
    did = jnp.where(valid, (first - 4) * 4 + (second - 4), 0)
    col = lax.broadcasted_iota(jnp.int32, (B, S), 1)
    did = jnp.where(col == S - 1, 16, did)
    c_ref[...] = first * 32 + did


def _prep(input_ids, token_table, dinuc_table, proj_w, proj_b, ln_gamma,
          ln_beta):
    return pl.pallas_call(
        _prep_body,
        out_shape=(
            jax.ShapeDtypeStruct((8, 32, D), jnp.float32),
            jax.ShapeDtypeStruct((B, S), jnp.int32),
        ),
    )(input_ids, token_table, dinuc_table, proj_w,
      proj_b.reshape(1, D), ln_gamma.reshape(1, D), ln_beta.reshape(1, D))


PW = D // 2
LPITCH = PW + 1
NROW = 8 * 32
TT = TOK // NW
GT = 16
NG = TT // GT
OPITCH = D + 1
NCHAIN = 8


def _fill_block(idx_v, lut_v, ob, g, phase, iota16):
    c_vec = idx_v[pl.ds(g * GT, GT)]
    row_vec = phase * GT + iota16
    se = [jnp.full((GT,), 2 * b, jnp.int32) for b in range(NCHAIN)]
    so = [jnp.full((GT,), 2 * b + 1, jnp.int32) for b in range(NCHAIN)]
    la = [jnp.full((GT,), b, jnp.int32) for b in range(NCHAIN)]
    dla = jnp.full((GT,), NCHAIN, jnp.int32)
    dse = jnp.full((GT,), 2 * NCHAIN, jnp.int32)
    for _ in range(PW // NCHAIN):
        for b in range(NCHAIN):
            w = plsc.load_gather(lut_v, [c_vec, la[b]])
            p0, p1 = plsc.unpack(plsc.bitcast(w, jnp.bfloat16),
                                 format=plsc.PackFormat.INTERLEAVED)
            plsc.store_scatter(ob, [row_vec, se[b]], p0)
            plsc.store_scatter(ob, [row_vec, so[b]], p1)
        for b in range(NCHAIN):
            la[b] = la[b] + dla
            se[b] = se[b] + dse
            so[b] = so[b] + dse


def _sc_gather_body(lut_hbm, idx_hbm, out_hbm, idx_v, lut_v, ob, sem0, sem1):
    wid = lax.axis_index("s") * NC + lax.axis_index("c")
    tok0 = wid * TT
    pltpu.sync_copy(lut_hbm, lut_v)
    pltpu.sync_copy(idx_hbm.at[pl.ds(tok0, TT)], idx_v)
    iota16 = lax.iota(jnp.int32, GT)

    def wcopy(i, phase, sem):
        return pltpu.make_async_copy(
            ob.at[pl.ds(phase * GT, GT), pl.ds(0, D)],
            out_hbm.at[pl.ds(tok0 + i * GT, GT)], sem)

    def body(i, carry):
        phase = lax.rem(i, 2)
        @pl.when(jnp.logical_and(i >= 2, phase == 0))
        def _():
            wcopy(i, 0, sem0).wait()
        @pl.when(jnp.logical_and(i >= 2, phase == 1))
        def _():
            wcopy(i, 1, sem1).wait()
        _fill_block(idx_v, lut_v, ob, i, phase, iota16)
        @pl.when(phase == 0)
        def _():
            wcopy(i, 0, sem0).start()
        @pl.when(phase == 1)
        def _():
            wcopy(i, 1, sem1).start()
        return carry

    lax.fori_loop(0, NG, body, 0)
    wcopy(0, 0, sem0).wait()
    wcopy(0, 1, sem1).wait()


@functools.cache
def _sc_gather():
    return pl.kernel(
        _sc_gather_body,
        out_type=jax.ShapeDtypeStruct((TOK, D), jnp.float32),
        mesh=plsc.VectorSubcoreMesh(core_axis_name="c", subcore_axis_name="s",
                                    num_cores=NC, num_subcores=NS),
        scratch_types=[
            pltpu.VMEM((TT,), jnp.int32),
            pltpu.VMEM((NROW, LPITCH), jnp.int32),
            pltpu.VMEM((2 * GT, OPITCH), jnp.float32),
            pltpu.SemaphoreType.DMA,
            pltpu.SemaphoreType.DMA,
        ],
        compiler_params=pltpu.CompilerParams(use_tc_tiling_on_sc=False,
                                             needs_layout_passes=False),
    )


@jax.jit
def kernel(input_ids, token_table, dinuc_table, proj_w, proj_b, ln_gamma,
           ln_beta):
    lut, c = _prep(input_ids, token_table, dinuc_table, proj_w, proj_b,
                   ln_gamma, ln_beta)
    lut_pk = lax.bitcast_convert_type(
        lut.astype(jnp.bfloat16).reshape(NROW, PW, 2), jnp.int32)
    lut_pk = jnp.pad(lut_pk, ((0, 0), (0, LPITCH - PW)))
    out = _sc_gather()(lut_pk, c.reshape(TOK))
    return out.reshape(B, S, D)
